# baseline (speedup 1.0000x reference)
import functools

import jax
import jax.numpy as jnp
from jax import lax
from jax.experimental import pallas as pl
from jax.experimental.pallas import tpu as pltpu
from jax.experimental.pallas import tpu_sc as plsc

D = 64
VOCAB = 1000000
NW = 32
NBUF = 4
SCALE = 8.0
ABLK = 512


@functools.cache
def _build(batch: int, hist: int):
    assert batch == NW * 128
    nh = hist
    per_chunk = 128 * D

    def a_body(tt_ref, out_ref):
        t = tt_ref[...].T * SCALE
        out_ref[...] = jnp.concatenate([t[: ABLK // 2], t[ABLK // 2 :]], 1)

    stage_a = pl.pallas_call(
        a_body,
        grid=(pl.cdiv(VOCAB, ABLK),),
        in_specs=[pl.BlockSpec((D, ABLK), lambda j: (0, j))],
        out_specs=pl.BlockSpec((ABLK // 2, 128), lambda j: (j, 0)),
        out_shape=jax.ShapeDtypeStruct(
            (pl.cdiv(VOCAB, ABLK) * ABLK * D // 128, 128), jnp.float32
        ),
    )

    mesh = plsc.VectorSubcoreMesh(core_axis_name="c", subcore_axis_name="s")
    ngrp = nh // NBUF
    assert nh == ngrp * NBUF

    RING = 2 * NBUF
    assert nh % RING == 0

    @functools.partial(
        pl.kernel,
        out_type=jax.ShapeDtypeStruct((nh * NW, 128, D), jnp.float32),
        mesh=mesh,
        scratch_types=[
            pltpu.VMEM((nh, 128), jnp.int32),
            pltpu.VMEM((RING, 128, D), jnp.float32),
        ]
        + [pltpu.SemaphoreType.DMA] * (2 * RING),
        compiler_params=pltpu.CompilerParams(use_tc_tiling_on_sc=False),
    )
    def stage_b(xt_hbm, t2_hbm, out_hbm, idx_v, g_v, *sems):
        gsem = sems[:RING]
        ssem = sems[RING:]
        wid = lax.axis_index("s") * 2 + lax.axis_index("c")
        pltpu.sync_copy(xt_hbm.at[wid], idx_v)

        for b in range(NBUF):
            pltpu.async_copy(t2_hbm.at[idx_v.at[b]], g_v.at[b], gsem[b])

        def group(hg, carry):
            for b in range(RING):
                h = hg * RING + b
                bn = (b + NBUF) % RING
                pltpu.make_async_copy(
                    t2_hbm.at[idx_v.at[h]], g_v.at[b], gsem[b]
                ).wait()

                @pl.when(h >= NBUF)
                def _():
                    pltpu.make_async_copy(
                        g_v.at[bn], out_hbm.at[(h - NBUF) * NW + wid],
                        ssem[bn],
                    ).wait()

                @pl.when(h + NBUF < nh)
                def _():
                    pltpu.async_copy(
                        t2_hbm.at[idx_v.at[h + NBUF]], g_v.at[bn], gsem[bn]
                    )

                pltpu.async_copy(g_v.at[b], out_hbm.at[h * NW + wid], ssem[b])
            return carry

        lax.fori_loop(0, nh // RING, group, 0)

        for b in range(NBUF):
            h = nh - NBUF + b
            pltpu.make_async_copy(
                g_v.at[h % RING], out_hbm.at[h * NW + wid], ssem[h % RING]
            ).wait()

    def c_body(in_ref, out_ref):
        v = in_ref[...]
        out_ref[0] = jnp.concatenate([v[:, :D].T, v[:, D:].T], 1)

    stage_c = pl.pallas_call(
        c_body,
        grid=(nh, NW),
        in_specs=[
            pl.BlockSpec((D, 128), lambda h, bt: (h * NW + bt, 0))
        ],
        out_specs=pl.BlockSpec((1, D, 128), lambda h, bt: (h, 0, bt)),
        out_shape=jax.ShapeDtypeStruct((nh, D, batch), jnp.float32),
    )

    return stage_a, stage_b, stage_c


def kernel(x, table):
    batch, hist = x.shape
    stage_a, stage_b, stage_c = _build(batch, hist)
    t1 = stage_a(table.T)
    xq = ((x & ~511) | ((x & 255) << 1) | ((x >> 8) & 1)).astype(jnp.int32)
    xt = (
        xq.T.reshape(hist, NW, 2, D)
        .transpose(1, 0, 3, 2)
        .reshape(NW, hist, 128)
    )
    flat = stage_b(xt, t1.reshape(t1.shape[0] * 2, D))
    o3 = stage_c(flat.reshape(batch * hist * D // 128, 128))
    return o3.transpose(2, 0, 1)


# speedup vs baseline: 4.3789x; 4.3789x over previous
import functools

import jax
import jax.numpy as jnp
from jax import lax
from jax.experimental import pallas as pl
from jax.experimental.pallas import tpu as pltpu
from jax.experimental.pallas import tpu_sc as plsc

D = 64
VOCAB = 1000000
NW = 32
NBUF = 4
SCALE = 8.0
ABLK = 4096


@functools.cache
def _build(batch: int, hist: int):
    assert batch == NW * 128
    nh = hist
    per_chunk = 128 * D

    def dott(x, eye):
        return lax.dot_general(
            x, eye, (((0,), (0,)), ((), ())),
            preferred_element_type=jnp.float32,
        )

    def a_body(tt_ref, eye_ref, out_ref):
        t = dott(tt_ref[...] * SCALE, eye_ref[...])
        out_ref[...] = jnp.concatenate([t[: ABLK // 2], t[ABLK // 2 :]], 1)

    stage_a = pl.pallas_call(
        a_body,
        grid=(pl.cdiv(VOCAB, ABLK),),
        in_specs=[
            pl.BlockSpec((D, ABLK), lambda j: (0, j)),
            pl.BlockSpec((D, D), lambda j: (0, 0)),
        ],
        out_specs=pl.BlockSpec((ABLK // 2, 128), lambda j: (j, 0)),
        out_shape=jax.ShapeDtypeStruct(
            (pl.cdiv(VOCAB, ABLK) * ABLK * D // 128, 128), jnp.float32
        ),
    )

    mesh = plsc.VectorSubcoreMesh(core_axis_name="c", subcore_axis_name="s")
    ngrp = nh // NBUF
    assert nh == ngrp * NBUF

    RING = 2 * NBUF
    assert nh % RING == 0

    @functools.partial(
        pl.kernel,
        out_type=jax.ShapeDtypeStruct((nh * NW, 128, D), jnp.float32),
        mesh=mesh,
        scratch_types=[
            pltpu.VMEM((nh, 128), jnp.int32),
            pltpu.VMEM((RING, 128, D), jnp.float32),
        ]
        + [pltpu.SemaphoreType.DMA] * (2 * RING),
        compiler_params=pltpu.CompilerParams(use_tc_tiling_on_sc=False),
    )
    def stage_b(xt_hbm, t2_hbm, out_hbm, idx_v, g_v, *sems):
        gsem = sems[:RING]
        ssem = sems[RING:]
        wid = lax.axis_index("s") * 2 + lax.axis_index("c")
        pltpu.sync_copy(xt_hbm.at[wid], idx_v)

        for b in range(NBUF):
            pltpu.async_copy(t2_hbm.at[idx_v.at[b]], g_v.at[b], gsem[b])

        def group(hg, carry):
            for b in range(RING):
                h = hg * RING + b
                bn = (b + NBUF) % RING
                pltpu.make_async_copy(
                    t2_hbm.at[idx_v.at[h]], g_v.at[b], gsem[b]
                ).wait()

                @pl.when(h >= NBUF)
                def _():
                    pltpu.make_async_copy(
                        g_v.at[bn], out_hbm.at[(h - NBUF) * NW + wid],
                        ssem[bn],
                    ).wait()

                @pl.when(h + NBUF < nh)
                def _():
                    pltpu.async_copy(
                        t2_hbm.at[idx_v.at[h + NBUF]], g_v.at[bn], gsem[bn]
                    )

                pltpu.async_copy(g_v.at[b], out_hbm.at[h * NW + wid], ssem[b])
            return carry

        lax.fori_loop(0, nh // RING, group, 0)

        for b in range(NBUF):
            h = nh - NBUF + b
            pltpu.make_async_copy(
                g_v.at[h % RING], out_hbm.at[h * NW + wid], ssem[h % RING]
            ).wait()

    CQ = 8

    def c_body(in_ref, eye_ref, out_ref):
        eye = eye_ref[...]
        for i in range(CQ):
            v = in_ref[pl.ds(i * D, D), :]
            out_ref[0, :, pl.ds(i * 128, 128)] = jnp.concatenate(
                [dott(v[:, :D], eye), dott(v[:, D:], eye)], 1
            )

    stage_c = pl.pallas_call(
        c_body,
        grid=(nh, NW // CQ),
        in_specs=[
            pl.BlockSpec((CQ * D, 128), lambda h, q: (h * (NW // CQ) + q, 0)),
            pl.BlockSpec((D, D), lambda h, q: (0, 0)),
        ],
        out_specs=pl.BlockSpec(
            (1, D, CQ * 128), lambda h, q: (h, 0, q)
        ),
        out_shape=jax.ShapeDtypeStruct((nh, D, batch), jnp.float32),
    )

    return stage_a, stage_b, stage_c


def kernel(x, table):
    batch, hist = x.shape
    stage_a, stage_b, stage_c = _build(batch, hist)
    eye = jnp.eye(D, dtype=jnp.float32)
    t1 = stage_a(table.T, eye)
    hb = ABLK // 2
    xq = ((x & ~(ABLK - 1)) | ((x & (hb - 1)) << 1) | ((x // hb) & 1)).astype(
        jnp.int32
    )
    xt = (
        xq.T.reshape(hist, NW, 2, D)
        .transpose(1, 0, 3, 2)
        .reshape(NW, hist, 128)
    )
    flat = stage_b(xt, t1.reshape(t1.shape[0] * 2, D))
    o3 = stage_c(flat.reshape(batch * hist * D // 128, 128), eye)
    return o3.transpose(2, 0, 1)
